# baseline (speedup 1.0000x reference)
import functools

import jax
import jax.numpy as jnp
from jax import lax
from jax.experimental import pallas as pl
from jax.experimental.pallas import tpu as pltpu
from jax.experimental.pallas import tpu_sc as plsc

_N_IDX = 16384
_N_WORDS = 100000
_HIST_PAD = 100352
_D = 64
_LANES = 16
_NS = 16
_IPW = _N_IDX // _NS
_VPW = _IPW // _LANES
_NCHUNK = 8
_CHUNK = _IPW // _NCHUNK


def _histogram(words):
    mesh = plsc.VectorSubcoreMesh(
        core_axis_name="c", subcore_axis_name="s", num_cores=1
    )
    zchunk = _HIST_PAD // _NS

    @functools.partial(
        pl.kernel,
        out_type=jax.ShapeDtypeStruct((_HIST_PAD,), jnp.float32),
        mesh=mesh,
        scratch_types=[
            pltpu.VMEM((_IPW,), jnp.int32),
            pltpu.VMEM((_NCHUNK, _CHUNK), jnp.float32),
            pltpu.VMEM((_NCHUNK, _CHUNK), jnp.int32),
            pltpu.VMEM((zchunk,), jnp.float32),
            pltpu.VMEM_SHARED((_HIST_PAD,), jnp.float32),
            pltpu.SemaphoreType.DMA,
        ],
        compiler_params=pltpu.CompilerParams(
            needs_layout_passes=False, skip_device_barrier=True
        ),
    )
    def body(words_hbm, out_hbm, idx_v, vals_v, idxo_v, zero_v, hist_s, sem):
        cid = lax.axis_index("c")
        sid = lax.axis_index("s")

        @pl.when(cid == 0)
        def _():
            pltpu.sync_copy(words_hbm.at[pl.ds(sid * _IPW, _IPW)], idx_v)

            zero16 = jnp.zeros((_LANES,), jnp.float32)

            def zstep(i, carry):
                zero_v[pl.ds(i * _LANES, _LANES)] = zero16
                return carry

            lax.fori_loop(0, zchunk // _LANES, zstep, 0, unroll=8)
            pltpu.sync_copy(zero_v, hist_s.at[pl.ds(sid * zchunk, zchunk)])

            lanes = lax.iota(jnp.int32, _LANES)
            for c in range(_NCHUNK):
                for j in range(_CHUNK // _LANES):
                    v = c * _CHUNK + j * _LANES
                    iv = idx_v[pl.ds(v, _LANES)]
                    cnt, last = plsc.scan_count(iv)
                    iout = jnp.where(last, iv, _N_WORDS + lanes)
                    vals_v[c, pl.ds(j * _LANES, _LANES)] = cnt.astype(
                        jnp.float32
                    )
                    idxo_v[c, pl.ds(j * _LANES, _LANES)] = iout

            plsc.subcore_barrier()
            copies = [
                pltpu.async_copy(
                    vals_v.at[c], hist_s.at[idxo_v.at[c]], sem, add=True
                )
                for c in range(_NCHUNK)
            ]
            for cp in copies:
                cp.wait()
            plsc.subcore_barrier()

            @pl.when(sid == 0)
            def _export():
                pltpu.sync_copy(hist_s, out_hbm)

    return body(words)


def _sweep(hist, table_t, bias2):
    def body(h_ref, e_ref, b_ref, o_ref):
        k = pl.program_id(0)
        h = h_ref[pl.ds(0, _N_WORDS)].reshape(1, _N_WORDS)
        r = jax.lax.dot_general(
            h, e_ref[...], (((1,), (1,)), ((), ())),
            preferred_element_type=jnp.float32,
        )
        r8 = jnp.concatenate([r] * (_D // 8), axis=1) + b_ref[...]
        col = jax.lax.broadcasted_iota(jnp.int32, (1, _D), 1) >> 3
        o_ref[...] = jnp.where(col == k, r8, o_ref[...])

    return pl.pallas_call(
        body,
        grid=(_D // 8,),
        in_specs=[
            pl.BlockSpec((_HIST_PAD,), lambda k: (0,)),
            pl.BlockSpec((8, _N_WORDS), lambda k: (k, 0)),
            pl.BlockSpec((1, _D), lambda k: (0, 0)),
        ],
        out_specs=pl.BlockSpec((1, _D), lambda k: (0, 0)),
        out_shape=jax.ShapeDtypeStruct((1, _D), jnp.float32),
    )(hist, table_t, bias2)


def kernel(words, embedding, bias):
    hist = _histogram(words.astype(jnp.int32))
    return _sweep(hist, embedding.T, bias.reshape(1, _D))


# speedup vs baseline: 1.1032x; 1.1032x over previous
import functools

import jax
import jax.numpy as jnp
from jax import lax
from jax.experimental import pallas as pl
from jax.experimental.pallas import tpu as pltpu
from jax.experimental.pallas import tpu_sc as plsc

_N_IDX = 16384
_N_WORDS = 100000
_HIST_PAD = 100352
_D = 64
_LANES = 16
_NS = 16
_IPW = _N_IDX // _NS
_VPW = _IPW // _LANES
_NCHUNK = 8
_CHUNK = _IPW // _NCHUNK


def _histogram(words):
    mesh = plsc.VectorSubcoreMesh(
        core_axis_name="c", subcore_axis_name="s", num_cores=1
    )
    zchunk = _HIST_PAD // _NS

    @functools.partial(
        pl.kernel,
        out_type=jax.ShapeDtypeStruct((_HIST_PAD,), jnp.float32),
        mesh=mesh,
        scratch_types=[
            pltpu.VMEM((_IPW,), jnp.int32),
            pltpu.VMEM((_NCHUNK, _CHUNK), jnp.float32),
            pltpu.VMEM((_NCHUNK, _CHUNK), jnp.int32),
            pltpu.VMEM((zchunk,), jnp.float32),
            pltpu.VMEM_SHARED((_HIST_PAD,), jnp.float32),
            pltpu.SemaphoreType.DMA,
        ],
        compiler_params=pltpu.CompilerParams(needs_layout_passes=False),
    )
    def body(words_hbm, out_hbm, idx_v, vals_v, idxo_v, zero_v, hist_s, sem):
        cid = lax.axis_index("c")
        sid = lax.axis_index("s")

        @pl.when(cid == 0)
        def _():
            pltpu.sync_copy(words_hbm.at[pl.ds(sid * _IPW, _IPW)], idx_v)

            zero16 = jnp.zeros((_LANES,), jnp.float32)

            def zstep(i, carry):
                zero_v[pl.ds(i * _LANES, _LANES)] = zero16
                return carry

            lax.fori_loop(0, zchunk // _LANES, zstep, 0, unroll=8)
            pltpu.sync_copy(zero_v, hist_s.at[pl.ds(sid * zchunk, zchunk)])

            lanes = lax.iota(jnp.int32, _LANES)
            for c in range(_NCHUNK):
                for j in range(_CHUNK // _LANES):
                    v = c * _CHUNK + j * _LANES
                    iv = idx_v[pl.ds(v, _LANES)]
                    cnt, last = plsc.scan_count(iv)
                    iout = jnp.where(last, iv, _N_WORDS + lanes)
                    vals_v[c, pl.ds(j * _LANES, _LANES)] = cnt.astype(
                        jnp.float32
                    )
                    idxo_v[c, pl.ds(j * _LANES, _LANES)] = iout

            plsc.subcore_barrier()
            copies = [
                pltpu.async_copy(
                    vals_v.at[c], hist_s.at[idxo_v.at[c]], sem, add=True
                )
                for c in range(_NCHUNK)
            ]
            for cp in copies:
                cp.wait()
            plsc.subcore_barrier()

            @pl.when(sid == 0)
            def _export():
                pltpu.sync_copy(hist_s, out_hbm)

    return body(words)


def _sweep(hist, table_t, bias2):
    def body(h_ref, e_ref, b_ref, o_ref):
        k = pl.program_id(0)
        h = h_ref[pl.ds(0, _N_WORDS)].reshape(1, _N_WORDS)
        r = jax.lax.dot_general(
            h, e_ref[...], (((1,), (1,)), ((), ())),
            preferred_element_type=jnp.float32,
        )
        r8 = jnp.concatenate([r] * (_D // 16), axis=1) + b_ref[...]
        col = jax.lax.broadcasted_iota(jnp.int32, (1, _D), 1) >> 4
        o_ref[...] = jnp.where(col == k, r8, o_ref[...])

    return pl.pallas_call(
        body,
        grid=(_D // 16,),
        in_specs=[
            pl.BlockSpec((_HIST_PAD,), lambda k: (0,)),
            pl.BlockSpec((16, _N_WORDS), lambda k: (k, 0)),
            pl.BlockSpec((1, _D), lambda k: (0, 0)),
        ],
        out_specs=pl.BlockSpec((1, _D), lambda k: (0, 0)),
        out_shape=jax.ShapeDtypeStruct((1, _D), jnp.float32),
    )(hist, table_t, bias2)


def kernel(words, embedding, bias):
    hist = _histogram(words.astype(jnp.int32))
    return _sweep(hist, embedding.T, bias.reshape(1, _D))
